# baseline (speedup 1.0000x reference)
import functools

import jax
import jax.numpy as jnp
from jax import lax
from jax.experimental import pallas as pl
from jax.experimental.pallas import tpu as pltpu
from jax.experimental.pallas import tpu_sc as plsc

NUM_ENTITIES = 1000000
EMBEDDING_DIM = 64
BATCH = 16384

_NUM_CORES = 2
_NUM_SUBCORES = 16
_NUM_WORKERS = _NUM_CORES * _NUM_SUBCORES
_B_PER_W = BATCH // _NUM_WORKERS
_OUT_DIM = 2 * EMBEDDING_DIM

_mesh = plsc.VectorSubcoreMesh(core_axis_name="c", subcore_axis_name="s")


@functools.partial(
    pl.kernel,
    mesh=_mesh,
    out_type=jax.ShapeDtypeStruct((BATCH, _OUT_DIM), jnp.float32),
    scratch_types=[
        pltpu.VMEM((_B_PER_W,), jnp.int32),
        pltpu.VMEM((_B_PER_W, EMBEDDING_DIM), jnp.float32),
        pltpu.SemaphoreType.DMA,
    ],
    compiler_params=pltpu.CompilerParams(use_tc_tiling_on_sc=False),
)
def _gather_kernel(idx_hbm, table_hbm, out_hbm, idx_v, rows_v, sem):
    wid = lax.axis_index("s") * _NUM_CORES + lax.axis_index("c")
    base = wid * _B_PER_W
    pltpu.sync_copy(idx_hbm.at[pl.ds(base, _B_PER_W)], idx_v)
    pltpu.async_copy(table_hbm.at[idx_v], rows_v, sem).wait()
    pltpu.sync_copy(
        rows_v, out_hbm.at[pl.ds(base, _B_PER_W), pl.ds(0, EMBEDDING_DIM)]
    )


@jax.jit
def kernel(entities, entity_embeddings):
    out128 = _gather_kernel(entities, entity_embeddings)
    return out128[:, :EMBEDDING_DIM]


# speedup vs baseline: 1.7016x; 1.7016x over previous
import functools

import jax
import jax.numpy as jnp
from jax import lax
from jax.experimental import pallas as pl
from jax.experimental.pallas import tpu as pltpu
from jax.experimental.pallas import tpu_sc as plsc

NUM_ENTITIES = 1000000
EMBEDDING_DIM = 64
BATCH = 16384

_NUM_CORES = 2
_NUM_SUBCORES = 16
_NUM_WORKERS = _NUM_CORES * _NUM_SUBCORES
_B_PER_W = BATCH // _NUM_WORKERS

_mesh = plsc.VectorSubcoreMesh(core_axis_name="c", subcore_axis_name="s")


@functools.partial(
    pl.kernel,
    mesh=_mesh,
    out_type=jax.ShapeDtypeStruct((BATCH, EMBEDDING_DIM), jnp.float32),
    scratch_types=[
        pltpu.VMEM((_B_PER_W,), jnp.int32),
        pltpu.VMEM((_B_PER_W, EMBEDDING_DIM), jnp.float32),
        pltpu.SemaphoreType.DMA,
    ]
    + [pltpu.SemaphoreType.DMA] * 8,
    compiler_params=pltpu.CompilerParams(use_tc_tiling_on_sc=True),
)
def _gather_kernel(idx_hbm, table_hbm, out_hbm, idx_v, rows_v, sem, *rsems):
    wid = lax.axis_index("s") * _NUM_CORES + lax.axis_index("c")
    base = wid * _B_PER_W
    pltpu.sync_copy(idx_hbm.at[pl.ds(base, _B_PER_W)], idx_v)

    def fire(c, carry):
        vec = idx_v[pl.ds(c * 16, 16)]
        for l in range(16):
            i = vec[l]
            pltpu.async_copy(
                table_hbm.at[pl.ds(i, 1), :],
                rows_v.at[pl.ds(c * 16 + l, 1), :],
                rsems[l % 8],
            )
        return carry

    lax.fori_loop(0, _B_PER_W // 16, fire, 0)

    for q in range(8):
        pltpu.make_async_copy(
            table_hbm.at[pl.ds(0, _B_PER_W // 8), :],
            rows_v.at[pl.ds(q * (_B_PER_W // 8), _B_PER_W // 8), :],
            rsems[q],
        ).wait()

    pltpu.sync_copy(rows_v, out_hbm.at[pl.ds(base, _B_PER_W)])


@jax.jit
def kernel(entities, entity_embeddings):
    return _gather_kernel(entities, entity_embeddings)


# speedup vs baseline: 2.2426x; 1.3180x over previous
import functools

import jax
import jax.numpy as jnp
from jax import lax
from jax.experimental import pallas as pl
from jax.experimental.pallas import tpu as pltpu
from jax.experimental.pallas import tpu_sc as plsc

NUM_ENTITIES = 1000000
EMBEDDING_DIM = 64
BATCH = 16384

_NUM_CORES = 2
_NUM_SUBCORES = 16
_NUM_WORKERS = _NUM_CORES * _NUM_SUBCORES
_B_PER_W = BATCH // _NUM_WORKERS

_T_BLOCK = 12800


def _transpose_body(tabT_ref, out_ref):
    out_ref[...] = tabT_ref[...].T


_transpose_call = pl.pallas_call(
    _transpose_body,
    grid=((NUM_ENTITIES + _T_BLOCK - 1) // _T_BLOCK,),
    in_specs=[
        pl.BlockSpec((EMBEDDING_DIM, _T_BLOCK), lambda i: (0, i)),
    ],
    out_specs=pl.BlockSpec((_T_BLOCK, EMBEDDING_DIM), lambda i: (i, 0)),
    out_shape=jax.ShapeDtypeStruct((NUM_ENTITIES, EMBEDDING_DIM), jnp.float32),
)

_mesh = plsc.VectorSubcoreMesh(core_axis_name="c", subcore_axis_name="s")


@functools.partial(
    pl.kernel,
    mesh=_mesh,
    out_type=jax.ShapeDtypeStruct((BATCH, EMBEDDING_DIM), jnp.float32),
    scratch_types=[
        pltpu.VMEM((_B_PER_W,), jnp.int32),
        pltpu.VMEM((_B_PER_W, EMBEDDING_DIM), jnp.float32),
        pltpu.SemaphoreType.DMA,
    ]
    + [pltpu.SemaphoreType.DMA] * 8,
    compiler_params=pltpu.CompilerParams(use_tc_tiling_on_sc=True),
)
def _gather_kernel(idx_hbm, table_hbm, out_hbm, idx_v, rows_v, sem, *rsems):
    wid = lax.axis_index("s") * _NUM_CORES + lax.axis_index("c")
    base = wid * _B_PER_W
    pltpu.sync_copy(idx_hbm.at[pl.ds(base, _B_PER_W)], idx_v)

    def fire(c, carry):
        vec = idx_v[pl.ds(c * 16, 16)]
        for l in range(16):
            i = vec[l]
            pltpu.async_copy(
                table_hbm.at[pl.ds(i, 1), :],
                rows_v.at[pl.ds(c * 16 + l, 1), :],
                rsems[l % 8],
            )
        return carry

    lax.fori_loop(0, _B_PER_W // 16, fire, 0)

    for q in range(8):
        pltpu.make_async_copy(
            table_hbm.at[pl.ds(0, _B_PER_W // 8), :],
            rows_v.at[pl.ds(q * (_B_PER_W // 8), _B_PER_W // 8), :],
            rsems[q],
        ).wait()

    pltpu.sync_copy(rows_v, out_hbm.at[pl.ds(base, _B_PER_W)])


@jax.jit
def kernel(entities, entity_embeddings):
    table_rm = _transpose_call(entity_embeddings.T)
    return _gather_kernel(entities, table_rm)


# speedup vs baseline: 2.3087x; 1.0294x over previous
import functools

import jax
import jax.numpy as jnp
from jax import lax
from jax.experimental import pallas as pl
from jax.experimental.pallas import tpu as pltpu
from jax.experimental.pallas import tpu_sc as plsc

NUM_ENTITIES = 1000000
EMBEDDING_DIM = 64
BATCH = 16384

_NUM_CORES = 2
_NUM_SUBCORES = 16
_NUM_WORKERS = _NUM_CORES * _NUM_SUBCORES
_B_PER_W = BATCH // _NUM_WORKERS

_T_BLOCK = 25600


def _transpose_body(tabT_ref, out_ref):
    out_ref[...] = tabT_ref[...].T


_transpose_call = pl.pallas_call(
    _transpose_body,
    grid=((NUM_ENTITIES + _T_BLOCK - 1) // _T_BLOCK,),
    in_specs=[
        pl.BlockSpec((EMBEDDING_DIM, _T_BLOCK), lambda i: (0, i)),
    ],
    out_specs=pl.BlockSpec((_T_BLOCK, EMBEDDING_DIM), lambda i: (i, 0)),
    out_shape=jax.ShapeDtypeStruct((NUM_ENTITIES, EMBEDDING_DIM), jnp.float32),
)

_mesh = plsc.VectorSubcoreMesh(core_axis_name="c", subcore_axis_name="s")


@functools.partial(
    pl.kernel,
    mesh=_mesh,
    out_type=jax.ShapeDtypeStruct((BATCH, EMBEDDING_DIM), jnp.float32),
    scratch_types=[
        pltpu.VMEM((_B_PER_W,), jnp.int32),
        pltpu.VMEM((_B_PER_W, EMBEDDING_DIM), jnp.float32),
        pltpu.SemaphoreType.DMA,
    ]
    + [pltpu.SemaphoreType.DMA] * 8,
    compiler_params=pltpu.CompilerParams(use_tc_tiling_on_sc=True),
)
def _gather_kernel(idx_hbm, table_hbm, out_hbm, idx_v, rows_v, sem, *rsems):
    wid = lax.axis_index("s") * _NUM_CORES + lax.axis_index("c")
    base = wid * _B_PER_W
    pltpu.sync_copy(idx_hbm.at[pl.ds(base, _B_PER_W)], idx_v)

    def fire(c, carry):
        vec = idx_v[pl.ds(c * 16, 16)]
        for l in range(16):
            i = vec[l]
            pltpu.async_copy(
                table_hbm.at[pl.ds(i, 1), :],
                rows_v.at[pl.ds(c * 16 + l, 1), :],
                rsems[l % 8],
            )
        return carry

    lax.fori_loop(0, _B_PER_W // 16, fire, 0)

    for q in range(8):
        pltpu.make_async_copy(
            table_hbm.at[pl.ds(0, _B_PER_W // 8), :],
            rows_v.at[pl.ds(q * (_B_PER_W // 8), _B_PER_W // 8), :],
            rsems[q],
        ).wait()

    pltpu.sync_copy(rows_v, out_hbm.at[pl.ds(base, _B_PER_W)])


@jax.jit
def kernel(entities, entity_embeddings):
    table_rm = _transpose_call(entity_embeddings.T)
    return _gather_kernel(entities, table_rm)


# speedup vs baseline: 2.3286x; 1.0086x over previous
import functools

import jax
import jax.numpy as jnp
from jax import lax
from jax.experimental import pallas as pl
from jax.experimental.pallas import tpu as pltpu
from jax.experimental.pallas import tpu_sc as plsc

NUM_ENTITIES = 1000000
EMBEDDING_DIM = 64
BATCH = 16384

_NUM_CORES = 2
_NUM_SUBCORES = 16
_NUM_WORKERS = _NUM_CORES * _NUM_SUBCORES
_B_PER_W = BATCH // _NUM_WORKERS

_T_BLOCK = 38400


def _transpose_body(tabT_ref, out_ref):
    out_ref[...] = tabT_ref[...].T


_transpose_call = pl.pallas_call(
    _transpose_body,
    grid=((NUM_ENTITIES + _T_BLOCK - 1) // _T_BLOCK,),
    in_specs=[
        pl.BlockSpec((EMBEDDING_DIM, _T_BLOCK), lambda i: (0, i)),
    ],
    out_specs=pl.BlockSpec((_T_BLOCK, EMBEDDING_DIM), lambda i: (i, 0)),
    out_shape=jax.ShapeDtypeStruct((NUM_ENTITIES, EMBEDDING_DIM), jnp.float32),
    compiler_params=pltpu.CompilerParams(vmem_limit_bytes=63 * 1024 * 1024),
)

_mesh = plsc.VectorSubcoreMesh(core_axis_name="c", subcore_axis_name="s")


@functools.partial(
    pl.kernel,
    mesh=_mesh,
    out_type=jax.ShapeDtypeStruct((BATCH, EMBEDDING_DIM), jnp.float32),
    scratch_types=[
        pltpu.VMEM((_B_PER_W,), jnp.int32),
        pltpu.VMEM((_B_PER_W, EMBEDDING_DIM), jnp.float32),
        pltpu.SemaphoreType.DMA,
    ]
    + [pltpu.SemaphoreType.DMA] * 8,
    compiler_params=pltpu.CompilerParams(use_tc_tiling_on_sc=True),
)
def _gather_kernel(idx_hbm, table_hbm, out_hbm, idx_v, rows_v, sem, *rsems):
    wid = lax.axis_index("s") * _NUM_CORES + lax.axis_index("c")
    base = wid * _B_PER_W
    pltpu.sync_copy(idx_hbm.at[pl.ds(base, _B_PER_W)], idx_v)

    def fire(c, carry):
        vec = idx_v[pl.ds(c * 16, 16)]
        for l in range(16):
            i = vec[l]
            pltpu.async_copy(
                table_hbm.at[pl.ds(i, 1), :],
                rows_v.at[pl.ds(c * 16 + l, 1), :],
                rsems[l % 8],
            )
        return carry

    lax.fori_loop(0, _B_PER_W // 16, fire, 0)

    for q in range(8):
        pltpu.make_async_copy(
            table_hbm.at[pl.ds(0, _B_PER_W // 8), :],
            rows_v.at[pl.ds(q * (_B_PER_W // 8), _B_PER_W // 8), :],
            rsems[q],
        ).wait()

    pltpu.sync_copy(rows_v, out_hbm.at[pl.ds(base, _B_PER_W)])


@jax.jit
def kernel(entities, entity_embeddings):
    table_rm = _transpose_call(entity_embeddings.T)
    return _gather_kernel(entities, table_rm)
